# baseline (speedup 1.0000x reference)
import functools

import jax
import jax.numpy as jnp
from jax import lax
from jax.experimental import pallas as pl
from jax.experimental.pallas import tpu as pltpu

_LAYERS = 3
_CH = 1024
_NV = 9
_T = 2048
_BLK = 512
_NW = 2 * _LAYERS + 1


def _lin(a, w, b):
    out = lax.dot_general(a, w, (((1,), (1,)), ((), ())),
                          preferred_element_type=jnp.float32)
    return out + b


def _body(x_ref, netw_hbm, netb_ref, rw_hbm, rb_ref, rwo_ref, rbo_ref,
          h_ref, routes_ref, idx_ref, wv, sems):
    pid = pl.program_id(0)

    @pl.when(pid == 0)
    def _start():
        for l in range(_LAYERS + 1):
            pltpu.make_async_copy(netw_hbm.at[l], wv.at[l], sems.at[l]).start()
        for l in range(_LAYERS):
            pltpu.make_async_copy(rw_hbm.at[l], wv.at[_LAYERS + 1 + l],
                                  sems.at[_LAYERS + 1 + l]).start()

    def _ready(slot):
        @pl.when(pid == 0)
        def _():
            pltpu.make_async_copy(netw_hbm.at[0], wv.at[slot],
                                  sems.at[slot]).wait()

    h = x_ref[...]
    for l in range(_LAYERS):
        _ready(l)
        h = _lin(h, wv[l], netb_ref[l])
        h = jnp.where(h >= 0, h, 0.2 * h)
    _ready(_LAYERS)
    h = _lin(h, wv[_LAYERS], netb_ref[_LAYERS])
    h_ref[...] = h

    g = h
    for l in range(_LAYERS):
        _ready(_LAYERS + 1 + l)
        g = _lin(g, wv[_LAYERS + 1 + l], rb_ref[l])
        g = jnp.where(g >= 0, g, 0.2 * g)
    logits = _lin(g, rwo_ref[...], rbo_ref[...])

    m = jnp.max(logits, axis=1, keepdims=True)
    e = jnp.exp(logits - m)
    routes = e / jnp.sum(e, axis=1, keepdims=True)
    routes_ref[...] = routes

    mx = jnp.max(routes, axis=1, keepdims=True)
    iot = lax.broadcasted_iota(jnp.int32, (_BLK, _NV), 1)
    idx = jnp.min(jnp.where(routes == mx, iot, _NV), axis=1, keepdims=True)
    idx_ref[...] = idx


@functools.partial(jax.jit)
def _run(xs, net_W, net_b, r_W, r_b, r_W_out, r_b_out2):
    grid = (_T // _BLK,)
    return pl.pallas_call(
        _body,
        grid=grid,
        in_specs=[
            pl.BlockSpec((_BLK, _CH), lambda i: (i, 0)),
            pl.BlockSpec(memory_space=pl.ANY),
            pl.BlockSpec((_LAYERS + 1, _CH), lambda i: (0, 0)),
            pl.BlockSpec(memory_space=pl.ANY),
            pl.BlockSpec((_LAYERS, _CH), lambda i: (0, 0)),
            pl.BlockSpec((_NV, _CH), lambda i: (0, 0)),
            pl.BlockSpec((1, _NV), lambda i: (0, 0)),
        ],
        out_specs=[
            pl.BlockSpec((_BLK, _CH), lambda i: (i, 0)),
            pl.BlockSpec((_BLK, _NV), lambda i: (i, 0)),
            pl.BlockSpec((_BLK, 1), lambda i: (i, 0)),
        ],
        out_shape=[
            jax.ShapeDtypeStruct((_T, _CH), jnp.float32),
            jax.ShapeDtypeStruct((_T, _NV), jnp.float32),
            jax.ShapeDtypeStruct((_T, 1), jnp.int32),
        ],
        scratch_shapes=[
            pltpu.VMEM((_NW, _CH, _CH), jnp.float32),
            pltpu.SemaphoreType.DMA((_NW,)),
        ],
    )(xs, net_W, net_b, r_W, r_b, r_W_out, r_b_out2)


def kernel(x, net_W, net_b, r_W, r_b, r_W_out, r_b_out):
    batch, time, channels = x.shape
    xs = x.reshape(time, channels)
    h, routes, idx = _run(xs, net_W, net_b, r_W, r_b, r_W_out,
                          r_b_out.reshape(1, -1))
    return h, routes, idx.reshape(time)


# speedup vs baseline: 1.0816x; 1.0816x over previous
import functools

import jax
import jax.numpy as jnp
from jax import lax
from jax.experimental import pallas as pl

_LAYERS = 3
_CH = 1024
_NV = 9
_T = 2048
_BLK = 1024


def _lin(a, w, b):
    out = lax.dot_general(a, w, (((1,), (1,)), ((), ())),
                          preferred_element_type=jnp.float32)
    return out + b


def _body(x_ref, netw_ref, netb_ref, rw_ref, rb_ref, rwo_ref, rbo_ref,
          h_ref, routes_ref, idx_ref):
    h = x_ref[...]
    for l in range(_LAYERS):
        h = _lin(h, netw_ref[l], netb_ref[l])
        h = jnp.where(h >= 0, h, 0.2 * h)
    h = _lin(h, netw_ref[_LAYERS], netb_ref[_LAYERS])
    h_ref[...] = h

    g = h
    for l in range(_LAYERS):
        g = _lin(g, rw_ref[l], rb_ref[l])
        g = jnp.where(g >= 0, g, 0.2 * g)
    logits = _lin(g, rwo_ref[...], rbo_ref[...])

    m = jnp.max(logits, axis=1, keepdims=True)
    e = jnp.exp(logits - m)
    routes = e / jnp.sum(e, axis=1, keepdims=True)
    routes_ref[...] = routes

    mx = jnp.max(routes, axis=1, keepdims=True)
    iot = lax.broadcasted_iota(jnp.int32, (_BLK, _NV), 1)
    idx = jnp.min(jnp.where(routes == mx, iot, _NV), axis=1, keepdims=True)
    idx_ref[...] = idx


@functools.partial(jax.jit)
def _run(xs, net_W, net_b, r_W, r_b, r_W_out, r_b_out2):
    grid = (_T // _BLK,)
    return pl.pallas_call(
        _body,
        grid=grid,
        in_specs=[
            pl.BlockSpec((_BLK, _CH), lambda i: (i, 0)),
            pl.BlockSpec((_LAYERS + 1, _CH, _CH), lambda i: (0, 0, 0)),
            pl.BlockSpec((_LAYERS + 1, _CH), lambda i: (0, 0)),
            pl.BlockSpec((_LAYERS, _CH, _CH), lambda i: (0, 0, 0)),
            pl.BlockSpec((_LAYERS, _CH), lambda i: (0, 0)),
            pl.BlockSpec((_NV, _CH), lambda i: (0, 0)),
            pl.BlockSpec((1, _NV), lambda i: (0, 0)),
        ],
        out_specs=[
            pl.BlockSpec((_BLK, _CH), lambda i: (i, 0)),
            pl.BlockSpec((_BLK, _NV), lambda i: (i, 0)),
            pl.BlockSpec((_BLK, 1), lambda i: (i, 0)),
        ],
        out_shape=[
            jax.ShapeDtypeStruct((_T, _CH), jnp.float32),
            jax.ShapeDtypeStruct((_T, _NV), jnp.float32),
            jax.ShapeDtypeStruct((_T, 1), jnp.int32),
        ],
    )(xs, net_W, net_b, r_W, r_b, r_W_out, r_b_out2)


def kernel(x, net_W, net_b, r_W, r_b, r_W_out, r_b_out):
    batch, time, channels = x.shape
    xs = x.reshape(time, channels)
    h, routes, idx = _run(xs, net_W, net_b, r_W, r_b, r_W_out,
                          r_b_out.reshape(1, -1))
    return h, routes, idx.reshape(time)
